# Initial kernel scaffold; baseline (speedup 1.0000x reference)
import jax
import jax.numpy as jnp
from jax.experimental import pallas as pl


def kernel(geom, feats):
    raise NotImplementedError("write your pallas kernel here")


# speedup vs baseline: 2.3861x; 2.3861x over previous
import functools

import jax
import jax.numpy as jnp
import numpy as np
from jax import lax
from jax.experimental import pallas as pl
from jax.experimental.pallas import tpu as pltpu
from jax.experimental.pallas import tpu_sc as plsc

B, N, D, H, W, C = 1, 6, 40, 32, 88, 64
NX, NY, NZ = 360, 360, 1
NP = B * N * D * H * W
NB = NX * NY
HB = NB // 2
NDUMP = 192
ACCR = HB + NDUMP
DUMP_MOD = 176

NTILES = 16
PT = NP // NTILES
BLK = 1408
NBLK = PT // BLK
NIDX = 128
NCH = BLK // NIDX
LROWS = NP // NIDX

ZPT = ACCR // NTILES
OPT = HB // NTILES

_DXF = np.array([0.3, 0.3, 20.0], dtype=np.float32)
_BXF = np.array([-54.0 + 0.15, -54.0 + 0.15, -10.0 + 10.0], dtype=np.float32)
_OFFF = _BXF - _DXF / np.float32(2.0)
OFF_X, OFF_Y, OFF_Z = (float(v) for v in _OFFF)
DX_X, DX_Y, DX_Z = (float(v) for v in _DXF)


def _bev_pool_sc(xs_a, ys_a, zs_a, feats2):
    mesh = plsc.VectorSubcoreMesh(core_axis_name="c", subcore_axis_name="s")

    @functools.partial(
        pl.kernel,
        mesh=mesh,
        out_type=(jax.ShapeDtypeStruct((4, 2, HB, 16), jnp.float32),
                  jax.ShapeDtypeStruct((2, LROWS, NIDX), jnp.int32)),
        scratch_types=[
            pltpu.VMEM((BLK, 16), jnp.float32),
            pltpu.VMEM((BLK, 16), jnp.float32),
            pltpu.VMEM((NCH, NIDX), jnp.int32),
            pltpu.VMEM((NCH, NIDX), jnp.int32),
            pltpu.VMEM((BLK,), jnp.float32),
            pltpu.VMEM((BLK,), jnp.float32),
            pltpu.VMEM((BLK,), jnp.float32),
            pltpu.VMEM_SHARED((ACCR, 16), jnp.float32),
            pltpu.SemaphoreType.DMA,
            pltpu.SemaphoreType.DMA,
            pltpu.SemaphoreType.DMA,
        ],
        compiler_params=pltpu.CompilerParams(use_tc_tiling_on_sc=False),
    )
    def bev_kernel(xs_hbm, ys_hbm, zs_hbm, feats_hbm, out_hbm, lloc_hbm,
                   fva, fvb, l2a, l2b, xs, ys, zs, acc, sema, semb, semsc):
        core = lax.axis_index("c")
        sub = lax.axis_index("s")
        lane = lax.iota(jnp.int32, 16)
        zero16 = jnp.zeros((16,), jnp.float32)
        tile_base = sub * PT
        lrow_base = sub * (PT // NIDX)
        hb_base = core * HB

        def fv_src(b, k):
            base = tile_base + b * BLK
            return feats_hbm.at[pl.ds(base, BLK), pl.ds(16 * k, 16)]

        def l2_src(b):
            return lloc_hbm.at[core, pl.ds(lrow_base + b * NCH, NCH)]

        def issue_load(b, k, fv, l2, sem):
            pltpu.async_copy(fv_src(b, k), fv, sem)
            pltpu.async_copy(l2_src(b), l2, sem)

        def wait_load(b, k, fv, l2, sem):
            pltpu.make_async_copy(fv_src(b, k), fv, sem).wait()
            pltpu.make_async_copy(l2_src(b), l2, sem).wait()

        def scatter_block(fv, l2):
            hs = [pltpu.async_copy(fv.at[pl.ds(j * NIDX, NIDX)],
                                   acc.at[l2.at[j]], semsc, add=True)
                  for j in range(NCH)]
            for h in hs:
                h.wait()

        for k in range(4):
            plsc.subcore_barrier()

            def zfill(j, _):
                fva[j, :] = zero16
                return 0

            lax.fori_loop(0, BLK, zfill, 0)
            zbase = sub * ZPT
            pltpu.sync_copy(fva, acc.at[pl.ds(zbase, BLK)])
            pltpu.sync_copy(fva, acc.at[pl.ds(zbase + BLK, BLK)])
            pltpu.sync_copy(fva.at[pl.ds(0, ZPT - 2 * BLK)],
                            acc.at[pl.ds(zbase + 2 * BLK, ZPT - 2 * BLK)])
            plsc.subcore_barrier()

            if k == 0:
                def block0(b, _):
                    base = tile_base + b * BLK
                    hfv = pltpu.async_copy(fv_src(b, 0), fva, sema)
                    pltpu.sync_copy(xs_hbm.at[pl.ds(base, BLK)], xs)
                    pltpu.sync_copy(ys_hbm.at[pl.ds(base, BLK)], ys)
                    pltpu.sync_copy(zs_hbm.at[pl.ds(base, BLK)], zs)

                    def cbody(jj, _):
                        for i2 in range(8):
                            off = jj * NIDX + i2 * 16
                            x = xs[pl.ds(off, 16)]
                            y = ys[pl.ds(off, 16)]
                            z = zs[pl.ds(off, 16)]
                            xi = ((x - OFF_X) / DX_X).astype(jnp.int32)
                            yi = ((y - OFF_Y) / DX_Y).astype(jnp.int32)
                            zi = ((z - OFF_Z) / DX_Z).astype(jnp.int32)
                            kept = ((xi >= 0) & (xi < NX)
                                    & (yi >= 0) & (yi < NY)
                                    & (zi >= 0) & (zi < NZ))
                            loc = xi * NY + yi - hb_base
                            inr = kept & (loc >= 0) & (loc < HB)
                            dump = HB + (off % DUMP_MOD) + lane
                            l2a[jj, pl.ds(i2 * 16, 16)] = jnp.where(
                                inr, loc, dump)
                        return 0

                    lax.fori_loop(0, NCH, cbody, 0)
                    pltpu.sync_copy(l2a, l2_src(b))
                    hfv.wait()
                    scatter_block(fva, l2a)
                    return 0

                lax.fori_loop(0, NBLK, block0, 0)
            else:
                issue_load(0, k, fva, l2a, sema)
                issue_load(1, k, fvb, l2b, semb)

                def step(b, fv, l2, sem):
                    wait_load(b, k, fv, l2, sem)
                    scatter_block(fv, l2)

                    @pl.when(b + 2 < NBLK)
                    def _():
                        issue_load(b + 2, k, fv, l2, sem)

                def pair(i, _):
                    step(2 * i, fva, l2a, sema)
                    step(2 * i + 1, fvb, l2b, semb)
                    return 0

                lax.fori_loop(0, NBLK // 2, pair, 0)

            plsc.subcore_barrier()
            orow = sub * OPT
            pltpu.sync_copy(acc.at[pl.ds(orow, OPT)],
                            out_hbm.at[k, core, pl.ds(orow, OPT)])

    return bev_kernel(xs_a, ys_a, zs_a, feats2)


@jax.jit
def kernel(geom, feats):
    g = geom.reshape(NP, 3)
    feats2 = feats.reshape(NP, C)
    pooled, _ = _bev_pool_sc(g[:, 0], g[:, 1], g[:, 2], feats2)
    out = pooled.transpose(0, 3, 1, 2).reshape(1, C * NZ, NX, NY)
    return out
